# baseline (speedup 1.0000x reference)
import jax
import jax.numpy as jnp
from jax import lax
from jax.experimental import pallas as pl
from jax.experimental.pallas import tpu as pltpu
from jax.experimental.pallas import tpu_sc as plsc

NC = 2
NS = 16
NW = NC * NS
L = 16

ROWS = 128
COLS = 32768
ROWS_PER_W = ROWS // NW
VECS = COLS // L
NBUCK = 256
UNROLL = 8


def _mono_key(x):
    u = plsc.bitcast(x, jnp.uint32)
    s = u >> jnp.uint32(31)
    return u ^ ((jnp.uint32(0) - s) | jnp.uint32(0x80000000))


def _body(scores_hbm, kvec_hbm, out_hbm, row_a, row_b, hist_v, cand_v,
          kvec_v, in_sems, out_sems):
    wid = lax.axis_index("s") * NC + lax.axis_index("c")
    pltpu.sync_copy(kvec_hbm, kvec_v)
    k_scalar = jnp.sum(kvec_v[:]) >> 4
    iota16 = lax.iota(jnp.int32, L)
    ones16 = jnp.ones((L,), jnp.int32)
    zeros16 = jnp.zeros((L,), jnp.int32)
    rows = [row_a, row_b]
    base = wid * ROWS_PER_W

    def scan_hist(rem_in, prefix_in, shift):
        @plsc.parallel_loop(0, NBUCK, unroll=8,
                            carry=(rem_in, jnp.int32(0), jnp.bool_(False)))
        def _scn(b, carry):
            rem_c, chosen_c, done_c = carry
            d = NBUCK - 1 - b
            cnt = jnp.sum(hist_v[pl.ds(d * L, L)])
            stop = jnp.logical_and(jnp.logical_not(done_c), cnt >= rem_c)
            chosen_c = jnp.where(stop, d, chosen_c)
            keep_going = jnp.logical_not(jnp.logical_or(done_c, stop))
            rem_c = jnp.where(keep_going, rem_c - cnt, rem_c)
            return rem_c, chosen_c, jnp.logical_or(done_c, stop)

        rem_out, chosen, _done = _scn
        return rem_out, prefix_in | (chosen.astype(jnp.uint32) << shift)

    def clear_hist():
        @plsc.parallel_loop(0, NBUCK, unroll=8)
        def _clr(b):
            hist_v[pl.ds(b * L, L)] = zeros16

    for j in range(ROWS_PER_W):
        row_v = rows[j % 2]
        nxt_v = rows[(j + 1) % 2]
        if j == 0:
            pltpu.async_copy(scores_hbm.at[base], row_a, in_sems.at[0])
        pltpu.make_async_copy(scores_hbm.at[base + j], row_v,
                              in_sems.at[j % 2]).wait()
        if j + 1 < ROWS_PER_W:
            if j >= 1:
                pltpu.make_async_copy(
                    nxt_v, out_hbm.at[base + j - 1],
                    out_sems.at[(j + 1) % 2]).wait()
            pltpu.async_copy(scores_hbm.at[base + j + 1], nxt_v,
                             in_sems.at[(j + 1) % 2])

        clear_hist()

        @plsc.parallel_loop(0, VECS, unroll=UNROLL)
        def _dat0(i):
            key = _mono_key(row_v[pl.ds(i * L, L)])
            digit = ((key >> jnp.uint32(24)) & jnp.uint32(0xFF)
                     ).astype(jnp.int32)
            plsc.addupdate_scatter(hist_v, [digit * L + iota16], ones16)

        rem, prefix = scan_hist(k_scalar, jnp.uint32(0), jnp.uint32(24))

        clear_hist()
        m1 = jnp.uint32(0xFF000000)
        pfx1 = prefix & m1

        @plsc.parallel_loop(0, VECS, unroll=8, carry=jnp.int32(0))
        def _dat1(i, off):
            key = _mono_key(row_v[pl.ds(i * L, L)])
            digit = ((key >> jnp.uint32(16)) & jnp.uint32(0xFF)
                     ).astype(jnp.int32)
            match = (key & m1) == pfx1
            plsc.addupdate_scatter(
                hist_v, [digit * L + iota16], ones16, mask=match)
            plsc.store_compressed(cand_v.at[pl.ds(off, L)],
                                  plsc.bitcast(key, jnp.int32), mask=match)
            return off + jnp.sum(match.astype(jnp.int32))

        off = _dat1
        pad = (prefix ^ jnp.uint32(0xFF000000)).astype(jnp.int32)
        cand_v[pl.ds(off, L)] = jnp.full((L,), pad, jnp.int32)
        ncv = (off + jnp.int32(L - 1)) >> 4
        rem, prefix = scan_hist(rem, prefix, jnp.uint32(16))

        for lev in range(2, 4):
            shift = jnp.uint32(24 - 8 * lev)
            himask = jnp.uint32((0xFFFFFFFF << (32 - 8 * lev)) & 0xFFFFFFFF)
            pfx = prefix & himask
            clear_hist()

            @plsc.parallel_loop(0, ncv, unroll=1)
            def _datc(i):
                key = plsc.bitcast(cand_v[pl.ds(i * L, L)], jnp.uint32)
                digit = ((key >> shift) & jnp.uint32(0xFF)).astype(jnp.int32)
                match = (key & himask) == pfx
                plsc.addupdate_scatter(
                    hist_v, [digit * L + iota16], ones16, mask=match)

            rem, prefix = scan_hist(rem, prefix, shift)

        tkey = jnp.full((L,), prefix, jnp.uint32)
        tvec = plsc.bitcast(
            tkey ^ (((tkey >> jnp.uint32(31)) - jnp.uint32(1))
                    | jnp.uint32(0x80000000)), jnp.float32)
        rem_splat = jnp.full((L,), rem, jnp.int32)

        @plsc.parallel_loop(0, VECS, unroll=UNROLL, carry=zeros16)
        def _outp(i, cnt):
            sl = pl.ds(i * L, L)
            x = row_v[sl]
            gt = x > tvec
            eq = x == tvec
            inc = plsc.cumsum(eq.astype(jnp.int32))
            keep = jnp.logical_or(
                gt, jnp.logical_and(eq, (inc + cnt) <= rem_splat))
            row_v[sl] = jnp.where(keep, x, jnp.float32(0.0))
            return cnt + plsc.all_reduce_population_count(eq)

        pltpu.async_copy(row_v, out_hbm.at[base + j], out_sems.at[j % 2])

    pltpu.make_async_copy(rows[(ROWS_PER_W - 2) % 2],
                          out_hbm.at[base + ROWS_PER_W - 2],
                          out_sems.at[(ROWS_PER_W - 2) % 2]).wait()
    pltpu.make_async_copy(rows[(ROWS_PER_W - 1) % 2],
                          out_hbm.at[base + ROWS_PER_W - 1],
                          out_sems.at[(ROWS_PER_W - 1) % 2]).wait()


def kernel(scores, k):
    kvec = jnp.full((L,), k, jnp.int32)
    mesh = plsc.VectorSubcoreMesh(
        core_axis_name="c", subcore_axis_name="s",
        num_cores=NC, num_subcores=NS)
    fn = pl.kernel(
        _body,
        out_type=jax.ShapeDtypeStruct((ROWS, COLS), jnp.float32),
        mesh=mesh,
        scratch_types=[
            pltpu.VMEM((COLS,), jnp.float32),
            pltpu.VMEM((COLS,), jnp.float32),
            pltpu.VMEM((NBUCK * L,), jnp.int32),
            pltpu.VMEM((COLS + L,), jnp.int32),
            pltpu.VMEM((L,), jnp.int32),
            pltpu.SemaphoreType.DMA((2,)),
            pltpu.SemaphoreType.DMA((2,)),
        ],
        compiler_params=pltpu.CompilerParams(needs_layout_passes=False),
    )
    return fn(scores, kvec)


# speedup vs baseline: 1.0107x; 1.0107x over previous
import jax
import jax.numpy as jnp
from jax import lax
from jax.experimental import pallas as pl
from jax.experimental.pallas import tpu as pltpu
from jax.experimental.pallas import tpu_sc as plsc

NC = 2
NS = 16
NW = NC * NS
L = 16

ROWS = 128
COLS = 32768
ROWS_PER_W = ROWS // NW
VECS = COLS // L
NBUCK = 256
UNROLL = 8


def _mono_key(x):
    u = plsc.bitcast(x, jnp.uint32)
    s = u >> jnp.uint32(31)
    return u ^ ((jnp.uint32(0) - s) | jnp.uint32(0x80000000))


def _body(scores_hbm, kvec_hbm, out_hbm, row_a, row_b, hist_v, cand_v,
          kvec_v, in_sems, out_sems):
    wid = lax.axis_index("s") * NC + lax.axis_index("c")
    pltpu.sync_copy(kvec_hbm, kvec_v)
    k_scalar = jnp.sum(kvec_v[:]) >> 4
    iota16 = lax.iota(jnp.int32, L)
    ones16 = jnp.ones((L,), jnp.int32)
    zeros16 = jnp.zeros((L,), jnp.int32)
    rows = [row_a, row_b]
    base = wid * ROWS_PER_W

    def scan_hist(rem_in, prefix_in, shift):
        @plsc.parallel_loop(0, NBUCK, unroll=4,
                            carry=(rem_in, jnp.int32(0), jnp.bool_(False)))
        def _scn(b, carry):
            rem_c, chosen_c, done_c = carry
            d = NBUCK - 1 - b
            cnt = jnp.sum(hist_v[pl.ds(d * L, L)])
            stop = jnp.logical_and(jnp.logical_not(done_c), cnt >= rem_c)
            chosen_c = jnp.where(stop, d, chosen_c)
            keep_going = jnp.logical_not(jnp.logical_or(done_c, stop))
            rem_c = jnp.where(keep_going, rem_c - cnt, rem_c)
            return rem_c, chosen_c, jnp.logical_or(done_c, stop)

        rem_out, chosen, _done = _scn
        return rem_out, prefix_in | (chosen.astype(jnp.uint32) << shift)

    def clear_hist():
        @plsc.parallel_loop(0, NBUCK, unroll=8)
        def _clr(b):
            hist_v[pl.ds(b * L, L)] = zeros16

    for j in range(ROWS_PER_W):
        row_v = rows[j % 2]
        nxt_v = rows[(j + 1) % 2]
        if j == 0:
            pltpu.async_copy(scores_hbm.at[base], row_a, in_sems.at[0])
        pltpu.make_async_copy(scores_hbm.at[base + j], row_v,
                              in_sems.at[j % 2]).wait()
        if j + 1 < ROWS_PER_W:
            if j >= 1:
                pltpu.make_async_copy(
                    nxt_v, out_hbm.at[base + j - 1],
                    out_sems.at[(j + 1) % 2]).wait()
            pltpu.async_copy(scores_hbm.at[base + j + 1], nxt_v,
                             in_sems.at[(j + 1) % 2])

        clear_hist()

        @plsc.parallel_loop(0, VECS, unroll=UNROLL)
        def _dat0(i):
            key = _mono_key(row_v[pl.ds(i * L, L)])
            digit = ((key >> jnp.uint32(24)) & jnp.uint32(0xFF)
                     ).astype(jnp.int32)
            plsc.addupdate_scatter(hist_v, [digit * L + iota16], ones16)

        rem, prefix = scan_hist(k_scalar, jnp.uint32(0), jnp.uint32(24))

        clear_hist()
        m1 = jnp.uint32(0xFF000000)
        pfx1 = prefix & m1

        @plsc.parallel_loop(0, VECS, unroll=8, carry=jnp.int32(0))
        def _dat1(i, off):
            key = _mono_key(row_v[pl.ds(i * L, L)])
            digit = ((key >> jnp.uint32(16)) & jnp.uint32(0xFF)
                     ).astype(jnp.int32)
            match = (key & m1) == pfx1
            plsc.addupdate_scatter(
                hist_v, [digit * L + iota16], ones16, mask=match)
            plsc.store_compressed(cand_v.at[pl.ds(off, L)],
                                  plsc.bitcast(key, jnp.int32), mask=match)
            return off + jnp.sum(match.astype(jnp.int32))

        off = _dat1
        pad = (prefix ^ jnp.uint32(0xFF000000)).astype(jnp.int32)
        cand_v[pl.ds(off, L)] = jnp.full((L,), pad, jnp.int32)
        ncv = (off + jnp.int32(L - 1)) >> 4
        rem, prefix = scan_hist(rem, prefix, jnp.uint32(16))

        for lev in range(2, 4):
            shift = jnp.uint32(24 - 8 * lev)
            himask = jnp.uint32((0xFFFFFFFF << (32 - 8 * lev)) & 0xFFFFFFFF)
            pfx = prefix & himask
            clear_hist()

            @plsc.parallel_loop(0, ncv, unroll=1)
            def _datc(i):
                key = plsc.bitcast(cand_v[pl.ds(i * L, L)], jnp.uint32)
                digit = ((key >> shift) & jnp.uint32(0xFF)).astype(jnp.int32)
                match = (key & himask) == pfx
                plsc.addupdate_scatter(
                    hist_v, [digit * L + iota16], ones16, mask=match)

            rem, prefix = scan_hist(rem, prefix, shift)

        tkey = jnp.full((L,), prefix, jnp.uint32)
        tvec = plsc.bitcast(
            tkey ^ (((tkey >> jnp.uint32(31)) - jnp.uint32(1))
                    | jnp.uint32(0x80000000)), jnp.float32)
        rem_splat = jnp.full((L,), rem, jnp.int32)

        @plsc.parallel_loop(0, VECS, unroll=UNROLL, carry=zeros16)
        def _outp(i, cnt):
            sl = pl.ds(i * L, L)
            x = row_v[sl]
            gt = x > tvec
            eq = x == tvec
            inc = plsc.cumsum(eq.astype(jnp.int32))
            keep = jnp.logical_or(
                gt, jnp.logical_and(eq, (inc + cnt) <= rem_splat))
            row_v[sl] = jnp.where(keep, x, jnp.float32(0.0))
            return cnt + plsc.all_reduce_population_count(eq)

        pltpu.async_copy(row_v, out_hbm.at[base + j], out_sems.at[j % 2])

    pltpu.make_async_copy(rows[(ROWS_PER_W - 2) % 2],
                          out_hbm.at[base + ROWS_PER_W - 2],
                          out_sems.at[(ROWS_PER_W - 2) % 2]).wait()
    pltpu.make_async_copy(rows[(ROWS_PER_W - 1) % 2],
                          out_hbm.at[base + ROWS_PER_W - 1],
                          out_sems.at[(ROWS_PER_W - 1) % 2]).wait()


def kernel(scores, k):
    kvec = jnp.full((L,), k, jnp.int32)
    mesh = plsc.VectorSubcoreMesh(
        core_axis_name="c", subcore_axis_name="s",
        num_cores=NC, num_subcores=NS)
    fn = pl.kernel(
        _body,
        out_type=jax.ShapeDtypeStruct((ROWS, COLS), jnp.float32),
        mesh=mesh,
        scratch_types=[
            pltpu.VMEM((COLS,), jnp.float32),
            pltpu.VMEM((COLS,), jnp.float32),
            pltpu.VMEM((NBUCK * L,), jnp.int32),
            pltpu.VMEM((COLS + L,), jnp.int32),
            pltpu.VMEM((L,), jnp.int32),
            pltpu.SemaphoreType.DMA((2,)),
            pltpu.SemaphoreType.DMA((2,)),
        ],
        compiler_params=pltpu.CompilerParams(needs_layout_passes=False),
    )
    return fn(scores, kvec)


# speedup vs baseline: 1.0413x; 1.0303x over previous
import jax
import jax.numpy as jnp
from jax import lax
from jax.experimental import pallas as pl
from jax.experimental.pallas import tpu as pltpu
from jax.experimental.pallas import tpu_sc as plsc

NC = 2
NS = 16
NW = NC * NS
L = 16

ROWS = 128
COLS = 32768
ROWS_PER_W = ROWS // NW
VECS = COLS // L
NBUCK = 256
UNROLL = 8


def _mono_key(x):
    u = plsc.bitcast(x, jnp.uint32)
    s = u >> jnp.uint32(31)
    return u ^ ((jnp.uint32(0) - s) | jnp.uint32(0x80000000))


def _body(scores_hbm, kvec_hbm, out_hbm, row_a, row_b, hist_v, cand_v,
          kvec_v, in_sems, out_sems):
    wid = lax.axis_index("s") * NC + lax.axis_index("c")
    pltpu.sync_copy(kvec_hbm, kvec_v)
    k_scalar = jnp.sum(kvec_v[:]) >> 4
    iota16 = lax.iota(jnp.int32, L)
    ones16 = jnp.ones((L,), jnp.int32)
    zeros16 = jnp.zeros((L,), jnp.int32)
    rows = [row_a, row_b]
    base = wid * ROWS_PER_W

    def scan_hist(rem_in, prefix_in, shift):
        @plsc.parallel_loop(0, NBUCK // L, unroll=2,
                            carry=(rem_in, jnp.int32(0), jnp.bool_(False)))
        def _coarse(cc, carry):
            rem_c, chunk_c, done_c = carry
            c = NBUCK // L - 1 - cc
            t = hist_v[pl.ds(c * L * L, L)]
            for l in range(1, L):
                t = t + hist_v[pl.ds((c * L + l) * L, L)]
            s = jnp.sum(t)
            stop = jnp.logical_and(jnp.logical_not(done_c), s >= rem_c)
            chunk_c = jnp.where(stop, c, chunk_c)
            keep_going = jnp.logical_not(jnp.logical_or(done_c, stop))
            rem_c = jnp.where(keep_going, rem_c - s, rem_c)
            return rem_c, chunk_c, jnp.logical_or(done_c, stop)

        rem_mid, chunk, _d0 = _coarse

        @plsc.parallel_loop(0, L, unroll=4,
                            carry=(rem_mid, jnp.int32(0), jnp.bool_(False)))
        def _fine(b, carry):
            rem_c, chosen_c, done_c = carry
            d = chunk * L + (L - 1 - b)
            cnt = jnp.sum(hist_v[pl.ds(d * L, L)])
            stop = jnp.logical_and(jnp.logical_not(done_c), cnt >= rem_c)
            chosen_c = jnp.where(stop, d, chosen_c)
            keep_going = jnp.logical_not(jnp.logical_or(done_c, stop))
            rem_c = jnp.where(keep_going, rem_c - cnt, rem_c)
            return rem_c, chosen_c, jnp.logical_or(done_c, stop)

        rem_out, chosen, _done = _fine
        return rem_out, prefix_in | (chosen.astype(jnp.uint32) << shift)

    def clear_hist():
        @plsc.parallel_loop(0, NBUCK, unroll=8)
        def _clr(b):
            hist_v[pl.ds(b * L, L)] = zeros16

    for j in range(ROWS_PER_W):
        row_v = rows[j % 2]
        nxt_v = rows[(j + 1) % 2]
        if j == 0:
            pltpu.async_copy(scores_hbm.at[base], row_a, in_sems.at[0])
        pltpu.make_async_copy(scores_hbm.at[base + j], row_v,
                              in_sems.at[j % 2]).wait()
        if j + 1 < ROWS_PER_W:
            if j >= 1:
                pltpu.make_async_copy(
                    nxt_v, out_hbm.at[base + j - 1],
                    out_sems.at[(j + 1) % 2]).wait()
            pltpu.async_copy(scores_hbm.at[base + j + 1], nxt_v,
                             in_sems.at[(j + 1) % 2])

        clear_hist()

        @plsc.parallel_loop(0, VECS, unroll=UNROLL)
        def _dat0(i):
            key = _mono_key(row_v[pl.ds(i * L, L)])
            digit = ((key >> jnp.uint32(24)) & jnp.uint32(0xFF)
                     ).astype(jnp.int32)
            plsc.addupdate_scatter(hist_v, [digit * L + iota16], ones16)

        rem, prefix = scan_hist(k_scalar, jnp.uint32(0), jnp.uint32(24))

        clear_hist()
        m1 = jnp.uint32(0xFF000000)
        pfx1 = prefix & m1

        @plsc.parallel_loop(0, VECS, unroll=8, carry=jnp.int32(0))
        def _dat1(i, off):
            key = _mono_key(row_v[pl.ds(i * L, L)])
            digit = ((key >> jnp.uint32(16)) & jnp.uint32(0xFF)
                     ).astype(jnp.int32)
            match = (key & m1) == pfx1
            plsc.addupdate_scatter(
                hist_v, [digit * L + iota16], ones16, mask=match)
            plsc.store_compressed(cand_v.at[pl.ds(off, L)],
                                  plsc.bitcast(key, jnp.int32), mask=match)
            return off + jnp.sum(match.astype(jnp.int32))

        off = _dat1
        pad = (prefix ^ jnp.uint32(0xFF000000)).astype(jnp.int32)
        cand_v[pl.ds(off, L)] = jnp.full((L,), pad, jnp.int32)
        ncv = (off + jnp.int32(L - 1)) >> 4
        rem, prefix = scan_hist(rem, prefix, jnp.uint32(16))

        for lev in range(2, 4):
            shift = jnp.uint32(24 - 8 * lev)
            himask = jnp.uint32((0xFFFFFFFF << (32 - 8 * lev)) & 0xFFFFFFFF)
            pfx = prefix & himask
            clear_hist()

            @plsc.parallel_loop(0, ncv, unroll=1)
            def _datc(i):
                key = plsc.bitcast(cand_v[pl.ds(i * L, L)], jnp.uint32)
                digit = ((key >> shift) & jnp.uint32(0xFF)).astype(jnp.int32)
                match = (key & himask) == pfx
                plsc.addupdate_scatter(
                    hist_v, [digit * L + iota16], ones16, mask=match)

            rem, prefix = scan_hist(rem, prefix, shift)

        tkey = jnp.full((L,), prefix, jnp.uint32)
        tvec = plsc.bitcast(
            tkey ^ (((tkey >> jnp.uint32(31)) - jnp.uint32(1))
                    | jnp.uint32(0x80000000)), jnp.float32)
        rem_splat = jnp.full((L,), rem, jnp.int32)

        @plsc.parallel_loop(0, VECS, unroll=UNROLL, carry=zeros16)
        def _outp(i, cnt):
            sl = pl.ds(i * L, L)
            x = row_v[sl]
            gt = x > tvec
            eq = x == tvec
            inc = plsc.cumsum(eq.astype(jnp.int32))
            keep = jnp.logical_or(
                gt, jnp.logical_and(eq, (inc + cnt) <= rem_splat))
            row_v[sl] = jnp.where(keep, x, jnp.float32(0.0))
            return cnt + plsc.all_reduce_population_count(eq)

        pltpu.async_copy(row_v, out_hbm.at[base + j], out_sems.at[j % 2])

    pltpu.make_async_copy(rows[(ROWS_PER_W - 2) % 2],
                          out_hbm.at[base + ROWS_PER_W - 2],
                          out_sems.at[(ROWS_PER_W - 2) % 2]).wait()
    pltpu.make_async_copy(rows[(ROWS_PER_W - 1) % 2],
                          out_hbm.at[base + ROWS_PER_W - 1],
                          out_sems.at[(ROWS_PER_W - 1) % 2]).wait()


def kernel(scores, k):
    kvec = jnp.full((L,), k, jnp.int32)
    mesh = plsc.VectorSubcoreMesh(
        core_axis_name="c", subcore_axis_name="s",
        num_cores=NC, num_subcores=NS)
    fn = pl.kernel(
        _body,
        out_type=jax.ShapeDtypeStruct((ROWS, COLS), jnp.float32),
        mesh=mesh,
        scratch_types=[
            pltpu.VMEM((COLS,), jnp.float32),
            pltpu.VMEM((COLS,), jnp.float32),
            pltpu.VMEM((NBUCK * L,), jnp.int32),
            pltpu.VMEM((COLS + L,), jnp.int32),
            pltpu.VMEM((L,), jnp.int32),
            pltpu.SemaphoreType.DMA((2,)),
            pltpu.SemaphoreType.DMA((2,)),
        ],
        compiler_params=pltpu.CompilerParams(needs_layout_passes=False),
    )
    return fn(scores, kvec)
